# baseline (speedup 1.0000x reference)
import jax
import jax.numpy as jnp
from jax import lax
from jax.experimental import pallas as pl
from jax.experimental.pallas import tpu as pltpu
from jax.experimental.pallas import tpu_sc as plsc

E = 8
K = 2
D = 1024
N = 4096
C = 1024
A = K * N
NC, NS = 2, 16
NW = NC * NS
CHUNK = A // NW
WPE = C // CHUNK
GR = 64
NR = CHUNK // GR
TPB = N // NW
RT = 16
CRND = TPB // RT
NEG = -1e30


def _bc(s):
    return lax.broadcast(s, (16,))


def _gate_body(x_ref, wg_ref, ids_ref, vals_ref):
    lg = jnp.dot(x_ref[...], wg_ref[...], preferred_element_type=jnp.float32)
    col = lax.broadcasted_iota(jnp.int32, lg.shape, 1)
    lg = jnp.where(col < E, lg, NEG)
    m1 = jnp.max(lg, axis=1, keepdims=True)
    i1 = jnp.min(jnp.where(lg >= m1, col, 128), axis=1, keepdims=True)
    lg2 = jnp.where(col == i1, NEG, lg)
    m2 = jnp.max(lg2, axis=1, keepdims=True)
    i2 = jnp.min(jnp.where(lg2 >= m2, col, 128), axis=1, keepdims=True)
    z = jnp.sum(jnp.where(col < E, jnp.exp(lg - m1), 0.0), axis=1, keepdims=True)
    v1 = 1.0 / z
    v2 = jnp.exp(m2 - m1) / z
    ids_ref[...] = jnp.concatenate([i1, i2], axis=1).T
    vals_ref[...] = jnp.concatenate([v1, v2], axis=1).T


_gate = pl.pallas_call(
    _gate_body,
    out_shape=[
        jax.ShapeDtypeStruct((2, N), jnp.int32),
        jax.ShapeDtypeStruct((2, N), jnp.float32),
    ],
)


def _route_body(ids_hbm, vals_hbm, x_hbm, disp_hbm, dest_hbm, gates_hbm,
                ids_v, vals_v, dest_v, gates_v, src_v, rows_a, rows_b, hist_v,
                gsem, wsem):
    wid = lax.axis_index("s") * NC + lax.axis_index("c")
    base = wid * CHUNK
    pltpu.sync_copy(ids_hbm, ids_v)
    pltpu.sync_copy(vals_hbm.at[pl.ds(base, CHUNK)], vals_v)
    iota = lax.iota(jnp.int32, 16)

    hist_v[pl.ds(0, 16)] = jnp.zeros(16, jnp.int32)
    ones = jnp.full((16,), 1, jnp.int32)

    def _pref(v, _):
        plsc.addupdate_scatter(hist_v, [ids_v[pl.ds(v * 16, 16)]], ones)
        return 0

    lax.fori_loop(0, wid * 16, _pref, 0)
    hv = hist_v[pl.ds(0, 16)]
    h = [jnp.sum(jnp.where(iota == e, hv, 0)) for e in range(E)]
    for v in range(CHUNK // 16):
        idsv = ids_v[pl.ds(base + v * 16, 16)]
        loc = jnp.zeros(16, jnp.int32)
        for e in range(E):
            m = idsv == e
            mi = jnp.where(m, 1, 0)
            cs = plsc.cumsum(mi)
            loc = jnp.where(m, cs - 1 + _bc(h[e]), loc)
            h[e] = h[e] + jnp.sum(mi)
        within = loc < C
        gates_v[pl.ds(v * 16, 16)] = jnp.where(within, vals_v[pl.ds(v * 16, 16)], 0.0)
        dest_v[pl.ds(v * 16, 16)] = idsv * C + jnp.minimum(loc, C - 1)
    pltpu.sync_copy(dest_v, dest_hbm.at[pl.ds(base, CHUNK)])
    pltpu.sync_copy(gates_v, gates_hbm.at[pl.ds(base, CHUNK)])

    em = wid // WPE
    lo = (wid % WPE) * CHUNK
    for v in range(CHUNK // 16):
        src_v[pl.ds(v * 16, 16)] = jnp.zeros(16, jnp.int32)

    def _slots(v, cnt):
        idsv = ids_v[pl.ds(v * 16, 16)]
        m = idsv == _bc(em)
        mi = jnp.where(m, 1, 0)
        pos = _bc(cnt) + plsc.cumsum(mi) - 1
        sel = m & (pos >= _bc(lo)) & (pos < _bc(lo + CHUNK))
        tok = (_bc(v * 16) + iota) & (N - 1)
        idx = jnp.clip(pos - lo, 0, CHUNK - 1)
        plsc.store_scatter(src_v, [idx], tok, mask=sel)
        return cnt + jnp.sum(mi)

    lax.fori_loop(0, A // 16, _slots, jnp.int32(0))

    slot0 = em * C + lo
    bufs = [rows_a, rows_b]
    gd = [None] * NR
    wd = [None] * NR
    for r in range(min(2, NR)):
        gd[r] = pltpu.async_copy(x_hbm.at[src_v.at[pl.ds(r * GR, GR)]],
                                 bufs[r % 2], gsem)
    for r in range(NR):
        gd[r].wait()
        wd[r] = pltpu.async_copy(bufs[r % 2],
                                 disp_hbm.at[pl.ds(slot0 + r * GR, GR)], wsem)
        if r + 2 < NR:
            wd[r].wait()
            gd[r + 2] = pltpu.async_copy(x_hbm.at[src_v.at[pl.ds((r + 2) * GR, GR)]],
                                         bufs[r % 2], gsem)
    for r in range(max(0, NR - 2), NR):
        wd[r].wait()


_route = pl.kernel(
    _route_body,
    out_type=[
        jax.ShapeDtypeStruct((A, D // 2), jnp.int32),
        jax.ShapeDtypeStruct((A,), jnp.int32),
        jax.ShapeDtypeStruct((A,), jnp.float32),
    ],
    mesh=plsc.VectorSubcoreMesh(core_axis_name="c", subcore_axis_name="s",
                                num_cores=NC, num_subcores=NS),
    compiler_params=pltpu.CompilerParams(needs_layout_passes=False),
    scratch_types=[
        pltpu.VMEM((A,), jnp.int32),
        pltpu.VMEM((CHUNK,), jnp.float32),
        pltpu.VMEM((CHUNK,), jnp.int32),
        pltpu.VMEM((CHUNK,), jnp.float32),
        pltpu.VMEM((CHUNK,), jnp.int32),
        pltpu.VMEM((GR, D // 2), jnp.int32),
        pltpu.VMEM((GR, D // 2), jnp.int32),
        pltpu.VMEM((16,), jnp.int32),
        pltpu.SemaphoreType.DMA,
        pltpu.SemaphoreType.DMA,
    ],
)


CB = 512

def _ffn_body(disp_ref, w1_ref, b1_ref, w2_ref, b2_ref, y_ref):
    h = jnp.dot(disp_ref[0], w1_ref[0], preferred_element_type=jnp.float32)
    h = jnp.maximum(h + b1_ref[0], 0.0)
    y = jnp.dot(h.astype(jnp.bfloat16), w2_ref[0],
                preferred_element_type=jnp.float32)
    y_ref[0] = y + b2_ref[0]


_ffn = pl.pallas_call(
    _ffn_body,
    grid=(E, C // CB),
    in_specs=[
        pl.BlockSpec((1, CB, D), lambda e, c: (e, c, 0)),
        pl.BlockSpec((1, D, D), lambda e, c: (e, 0, 0)),
        pl.BlockSpec((1, 1, D), lambda e, c: (e, 0, 0)),
        pl.BlockSpec((1, D, D), lambda e, c: (e, 0, 0)),
        pl.BlockSpec((1, 1, D), lambda e, c: (e, 0, 0)),
    ],
    out_specs=pl.BlockSpec((1, CB, D), lambda e, c: (e, c, 0)),
    out_shape=jax.ShapeDtypeStruct((E, C, D), jnp.float32),
)


def _combine_body(y_hbm, dest_hbm, gates_hbm, out_hbm,
                  idx_v, g_v, r0a, r1a, r0b, r1b, oba, obb, gsem, wsem):
    wid = lax.axis_index("s") * NC + lax.axis_index("c")
    iota = lax.iota(jnp.int32, 16)
    base0 = wid * TPB
    pltpu.sync_copy(dest_hbm.at[pl.ds(base0, TPB)], idx_v.at[pl.ds(0, TPB)])
    pltpu.sync_copy(dest_hbm.at[pl.ds(N + base0, TPB)], idx_v.at[pl.ds(TPB, TPB)])
    pltpu.sync_copy(gates_hbm.at[pl.ds(base0, TPB)], g_v.at[pl.ds(0, TPB)])
    pltpu.sync_copy(gates_hbm.at[pl.ds(N + base0, TPB)], g_v.at[pl.ds(TPB, TPB)])

    r0 = [r0a, r0b]
    r1 = [r1a, r1b]
    obs = [oba, obb]
    gd0 = [None] * CRND
    gd1 = [None] * CRND
    wd = [None] * CRND

    def _fire(r):
        b = r % 2
        gd0[r] = pltpu.async_copy(y_hbm.at[idx_v.at[pl.ds(r * RT, RT)]], r0[b], gsem)
        gd1[r] = pltpu.async_copy(y_hbm.at[idx_v.at[pl.ds(TPB + r * RT, RT)]], r1[b], gsem)

    _fire(0)
    if CRND > 1:
        _fire(1)
    for r in range(CRND):
        b = r % 2
        gd0[r].wait()
        gd1[r].wait()
        if r >= 2:
            wd[r - 2].wait()
        rows0, rows1, ob = r0[b], r1[b], obs[b]
        g0c = g_v[pl.ds(r * RT, 16)]
        g1c = g_v[pl.ds(TPB + r * RT, 16)]

        def _tok(t, _):
            ln = _bc(t)
            g0v = _bc(jnp.sum(jnp.where(iota == ln, g0c, 0.0)))
            g1v = _bc(jnp.sum(jnp.where(iota == ln, g1c, 0.0)))

            def _j(j, __):
                ob[t, pl.ds(j * 16, 16)] = (g0v * rows0[t, pl.ds(j * 16, 16)]
                                            + g1v * rows1[t, pl.ds(j * 16, 16)])
                return 0

            lax.fori_loop(0, D // 16, _j, 0)
            return 0

        lax.fori_loop(0, RT, _tok, 0)
        wd[r] = pltpu.async_copy(ob, out_hbm.at[pl.ds(base0 + r * RT, RT)], wsem)
        if r + 2 < CRND:
            _fire(r + 2)
    for r in range(max(0, CRND - 2), CRND):
        wd[r].wait()


_combine = pl.kernel(
    _combine_body,
    out_type=jax.ShapeDtypeStruct((N, D), jnp.float32),
    mesh=plsc.VectorSubcoreMesh(core_axis_name="c", subcore_axis_name="s",
                                num_cores=NC, num_subcores=NS),
    compiler_params=pltpu.CompilerParams(needs_layout_passes=False),
    scratch_types=[
        pltpu.VMEM((2 * TPB,), jnp.int32),
        pltpu.VMEM((2 * TPB,), jnp.float32),
        pltpu.VMEM((RT, D), jnp.float32),
        pltpu.VMEM((RT, D), jnp.float32),
        pltpu.VMEM((RT, D), jnp.float32),
        pltpu.VMEM((RT, D), jnp.float32),
        pltpu.VMEM((RT, D), jnp.float32),
        pltpu.VMEM((RT, D), jnp.float32),
        pltpu.SemaphoreType.DMA,
        pltpu.SemaphoreType.DMA,
    ],
)


def kernel(input, wg, w1, b1, w2, b2):
    x = input.astype(jnp.float32)
    wgp = jnp.zeros((D, 128), jnp.float32).at[:, :E].set(wg.T)
    xbi = lax.bitcast_convert_type(
        x.astype(jnp.bfloat16).reshape(N, D // 2, 2), jnp.int32)
    w1b = w1.astype(jnp.bfloat16)
    w2b = w2.astype(jnp.bfloat16)
    ids2, vals2 = _gate(x, wgp)
    disp_i, dest, gates = _route(ids2.reshape(A), vals2.reshape(A), xbi)
    disp = lax.bitcast_convert_type(disp_i, jnp.bfloat16)
    y = _ffn(disp.reshape(E, C, D), w1b, b1, w2b, b2)
    out = _combine(y.reshape(A, D), dest, gates)
    return out


# speedup vs baseline: 2.2722x; 2.2722x over previous
import jax
import jax.numpy as jnp
from jax import lax
from jax.experimental import pallas as pl
from jax.experimental.pallas import tpu as pltpu
from jax.experimental.pallas import tpu_sc as plsc

E = 8
K = 2
D = 1024
N = 4096
C = 1024
A = K * N
NC, NS = 2, 16
NW = NC * NS
CHUNK = A // NW
WPE = C // CHUNK
GR = 32
NR = CHUNK // GR
TPB = N // NW
RT = 16
CRND = TPB // RT
NEG = -1e30


def _bc(s):
    return lax.broadcast(s, (16,))


def _gate_body(x_ref, wg_ref, ids_ref, vals_ref):
    lg = jnp.dot(x_ref[...], wg_ref[...], preferred_element_type=jnp.float32)
    col = lax.broadcasted_iota(jnp.int32, lg.shape, 1)
    lg = jnp.where(col < E, lg, NEG)
    m1 = jnp.max(lg, axis=1, keepdims=True)
    i1 = jnp.min(jnp.where(lg >= m1, col, 128), axis=1, keepdims=True)
    lg2 = jnp.where(col == i1, NEG, lg)
    m2 = jnp.max(lg2, axis=1, keepdims=True)
    i2 = jnp.min(jnp.where(lg2 >= m2, col, 128), axis=1, keepdims=True)
    z = jnp.sum(jnp.where(col < E, jnp.exp(lg - m1), 0.0), axis=1, keepdims=True)
    v1 = 1.0 / z
    v2 = jnp.exp(m2 - m1) / z
    ids_ref[...] = jnp.concatenate([i1, i2], axis=1).T
    vals_ref[...] = jnp.concatenate([v1, v2], axis=1).T


_gate = pl.pallas_call(
    _gate_body,
    out_shape=[
        jax.ShapeDtypeStruct((2, N), jnp.int32),
        jax.ShapeDtypeStruct((2, N), jnp.float32),
    ],
)


def _route_body(ids_hbm, vals_hbm, x_hbm, disp_hbm, dest_hbm, gates_hbm,
                ids_v, vals_v, dest_v, gates_v, src_v, rows_a, rows_b, hist_v,
                gsem, wsem):
    wid = lax.axis_index("s") * NC + lax.axis_index("c")
    base = wid * CHUNK
    pltpu.sync_copy(ids_hbm, ids_v)
    pltpu.sync_copy(vals_hbm.at[pl.ds(base, CHUNK)], vals_v)
    iota = lax.iota(jnp.int32, 16)

    hist_v[pl.ds(0, 16)] = jnp.zeros(16, jnp.int32)
    ones = jnp.full((16,), 1, jnp.int32)

    def _pref(v, _):
        plsc.addupdate_scatter(hist_v, [ids_v[pl.ds(v * 16, 16)]], ones)
        return 0

    lax.fori_loop(0, wid * 16, _pref, 0)
    hv = hist_v[pl.ds(0, 16)]
    h = [jnp.sum(jnp.where(iota == e, hv, 0)) for e in range(E)]
    for v in range(CHUNK // 16):
        idsv = ids_v[pl.ds(base + v * 16, 16)]
        loc = jnp.zeros(16, jnp.int32)
        for e in range(E):
            m = idsv == e
            mi = jnp.where(m, 1, 0)
            cs = plsc.cumsum(mi)
            loc = jnp.where(m, cs - 1 + _bc(h[e]), loc)
            h[e] = h[e] + jnp.sum(mi)
        within = loc < C
        gates_v[pl.ds(v * 16, 16)] = jnp.where(within, vals_v[pl.ds(v * 16, 16)], 0.0)
        dest_v[pl.ds(v * 16, 16)] = idsv * C + jnp.minimum(loc, C - 1)
    pltpu.sync_copy(dest_v, dest_hbm.at[pl.ds(base, CHUNK)])
    pltpu.sync_copy(gates_v, gates_hbm.at[pl.ds(base, CHUNK)])

    em = wid // WPE
    lo = (wid % WPE) * CHUNK
    for v in range(CHUNK // 16):
        src_v[pl.ds(v * 16, 16)] = jnp.zeros(16, jnp.int32)

    def _slots(v, cnt):
        idsv = ids_v[pl.ds(v * 16, 16)]
        m = idsv == _bc(em)
        mi = jnp.where(m, 1, 0)
        pos = _bc(cnt) + plsc.cumsum(mi) - 1
        sel = m & (pos >= _bc(lo)) & (pos < _bc(lo + CHUNK))
        tok = (_bc(v * 16) + iota) & (N - 1)
        idx = jnp.clip(pos - lo, 0, CHUNK - 1)
        plsc.store_scatter(src_v, [idx], tok, mask=sel)
        return cnt + jnp.sum(mi)

    lax.fori_loop(0, A // 16, _slots, jnp.int32(0))

    slot0 = em * C + lo
    bufs = [rows_a, rows_b]
    gd = [None] * NR
    wd = [None] * NR
    for r in range(min(2, NR)):
        gd[r] = pltpu.async_copy(x_hbm.at[src_v.at[pl.ds(r * GR, GR)]],
                                 bufs[r % 2], gsem)
    for r in range(NR):
        gd[r].wait()
        wd[r] = pltpu.async_copy(bufs[r % 2],
                                 disp_hbm.at[pl.ds(slot0 + r * GR, GR)], wsem)
        if r + 2 < NR:
            wd[r].wait()
            gd[r + 2] = pltpu.async_copy(x_hbm.at[src_v.at[pl.ds((r + 2) * GR, GR)]],
                                         bufs[r % 2], gsem)
    for r in range(max(0, NR - 2), NR):
        wd[r].wait()


_route = pl.kernel(
    _route_body,
    out_type=[
        jax.ShapeDtypeStruct((A, D), jnp.float32),
        jax.ShapeDtypeStruct((A,), jnp.int32),
        jax.ShapeDtypeStruct((A,), jnp.float32),
    ],
    mesh=plsc.VectorSubcoreMesh(core_axis_name="c", subcore_axis_name="s",
                                num_cores=NC, num_subcores=NS),
    compiler_params=pltpu.CompilerParams(needs_layout_passes=False),
    scratch_types=[
        pltpu.VMEM((A,), jnp.int32),
        pltpu.VMEM((CHUNK,), jnp.float32),
        pltpu.VMEM((CHUNK,), jnp.int32),
        pltpu.VMEM((CHUNK,), jnp.float32),
        pltpu.VMEM((CHUNK,), jnp.int32),
        pltpu.VMEM((GR, D), jnp.float32),
        pltpu.VMEM((GR, D), jnp.float32),
        pltpu.VMEM((16,), jnp.int32),
        pltpu.SemaphoreType.DMA,
        pltpu.SemaphoreType.DMA,
    ],
)


CB = 512

def _ffn_body(disp_ref, w1_ref, b1_ref, w2_ref, b2_ref, y_ref):
    a = disp_ref[0].astype(jnp.bfloat16)
    h = jnp.dot(a, w1_ref[0].astype(jnp.bfloat16), preferred_element_type=jnp.float32)
    h = jnp.maximum(h + b1_ref[0], 0.0)
    y = jnp.dot(h.astype(jnp.bfloat16), w2_ref[0].astype(jnp.bfloat16),
                preferred_element_type=jnp.float32)
    y_ref[0] = y + b2_ref[0]


_ffn = pl.pallas_call(
    _ffn_body,
    grid=(E, C // CB),
    in_specs=[
        pl.BlockSpec((1, CB, D), lambda e, c: (e, c, 0)),
        pl.BlockSpec((1, D, D), lambda e, c: (e, 0, 0)),
        pl.BlockSpec((1, 1, D), lambda e, c: (e, 0, 0)),
        pl.BlockSpec((1, D, D), lambda e, c: (e, 0, 0)),
        pl.BlockSpec((1, 1, D), lambda e, c: (e, 0, 0)),
    ],
    out_specs=pl.BlockSpec((1, CB, D), lambda e, c: (e, c, 0)),
    out_shape=jax.ShapeDtypeStruct((E, C, D), jnp.float32),
)


def _combine_body(y_hbm, dest_hbm, gates_hbm, out_hbm,
                  idx_v, g_v, r0a, r1a, r0b, r1b, oba, obb, gsem, wsem):
    wid = lax.axis_index("s") * NC + lax.axis_index("c")
    iota = lax.iota(jnp.int32, 16)
    base0 = wid * TPB
    pltpu.sync_copy(dest_hbm.at[pl.ds(base0, TPB)], idx_v.at[pl.ds(0, TPB)])
    pltpu.sync_copy(dest_hbm.at[pl.ds(N + base0, TPB)], idx_v.at[pl.ds(TPB, TPB)])
    pltpu.sync_copy(gates_hbm.at[pl.ds(base0, TPB)], g_v.at[pl.ds(0, TPB)])
    pltpu.sync_copy(gates_hbm.at[pl.ds(N + base0, TPB)], g_v.at[pl.ds(TPB, TPB)])

    r0 = [r0a, r0b]
    r1 = [r1a, r1b]
    obs = [oba, obb]
    gd0 = [None] * CRND
    gd1 = [None] * CRND
    wd = [None] * CRND

    def _fire(r):
        b = r % 2
        gd0[r] = pltpu.async_copy(y_hbm.at[idx_v.at[pl.ds(r * RT, RT)]], r0[b], gsem)
        gd1[r] = pltpu.async_copy(y_hbm.at[idx_v.at[pl.ds(TPB + r * RT, RT)]], r1[b], gsem)

    _fire(0)
    if CRND > 1:
        _fire(1)
    for r in range(CRND):
        b = r % 2
        gd0[r].wait()
        gd1[r].wait()
        if r >= 2:
            wd[r - 2].wait()
        rows0, rows1, ob = r0[b], r1[b], obs[b]
        g0c = g_v[pl.ds(r * RT, 16)]
        g1c = g_v[pl.ds(TPB + r * RT, 16)]

        def _tok(t, _):
            ln = _bc(t)
            g0v = _bc(jnp.sum(jnp.where(iota == ln, g0c, 0.0)))
            g1v = _bc(jnp.sum(jnp.where(iota == ln, g1c, 0.0)))

            def _j(j, __):
                ob[t, pl.ds(j * 16, 16)] = (g0v * rows0[t, pl.ds(j * 16, 16)]
                                            + g1v * rows1[t, pl.ds(j * 16, 16)])
                return 0

            lax.fori_loop(0, D // 16, _j, 0)
            return 0

        lax.fori_loop(0, RT, _tok, 0)
        wd[r] = pltpu.async_copy(ob, out_hbm.at[pl.ds(base0 + r * RT, RT)], wsem)
        if r + 2 < CRND:
            _fire(r + 2)
    for r in range(max(0, CRND - 2), CRND):
        wd[r].wait()


_combine = pl.kernel(
    _combine_body,
    out_type=jax.ShapeDtypeStruct((N, D), jnp.float32),
    mesh=plsc.VectorSubcoreMesh(core_axis_name="c", subcore_axis_name="s",
                                num_cores=NC, num_subcores=NS),
    compiler_params=pltpu.CompilerParams(needs_layout_passes=False),
    scratch_types=[
        pltpu.VMEM((2 * TPB,), jnp.int32),
        pltpu.VMEM((2 * TPB,), jnp.float32),
        pltpu.VMEM((RT, D), jnp.float32),
        pltpu.VMEM((RT, D), jnp.float32),
        pltpu.VMEM((RT, D), jnp.float32),
        pltpu.VMEM((RT, D), jnp.float32),
        pltpu.VMEM((RT, D), jnp.float32),
        pltpu.VMEM((RT, D), jnp.float32),
        pltpu.SemaphoreType.DMA,
        pltpu.SemaphoreType.DMA,
    ],
)


def kernel(input, wg, w1, b1, w2, b2):
    x = input.astype(jnp.float32)
    wgp = jnp.zeros((D, 128), jnp.float32).at[:, :E].set(wg.T)
    ids2, vals2 = _gate(x, wgp)
    disp, dest, gates = _route(ids2.reshape(A), vals2.reshape(A), x)
    y = _ffn(disp.reshape(E, C, D), w1, b1, w2, b2)
    out = _combine(y.reshape(A, D), dest, gates)
    return out


# speedup vs baseline: 2.4947x; 1.0979x over previous
import jax
import jax.numpy as jnp
from jax import lax
from jax.experimental import pallas as pl
from jax.experimental.pallas import tpu as pltpu
from jax.experimental.pallas import tpu_sc as plsc

E = 8
K = 2
D = 1024
N = 4096
C = 1024
A = K * N
NC, NS = 2, 16
NW = NC * NS
CHUNK = A // NW
WPE = C // CHUNK
GR = 32
NR = CHUNK // GR
TPB = N // NW
RT = 16
CRND = TPB // RT
NEG = -1e30


def _bc(s):
    return lax.broadcast(s, (16,))


def _gate_body(x_ref, wg_ref, ids_ref, vals_ref):
    lg = jnp.dot(x_ref[...], wg_ref[...], preferred_element_type=jnp.float32)
    col = lax.broadcasted_iota(jnp.int32, lg.shape, 1)
    lg = jnp.where(col < E, lg, NEG)
    m1 = jnp.max(lg, axis=1, keepdims=True)
    i1 = jnp.min(jnp.where(lg >= m1, col, 128), axis=1, keepdims=True)
    lg2 = jnp.where(col == i1, NEG, lg)
    m2 = jnp.max(lg2, axis=1, keepdims=True)
    i2 = jnp.min(jnp.where(lg2 >= m2, col, 128), axis=1, keepdims=True)
    z = jnp.sum(jnp.where(col < E, jnp.exp(lg - m1), 0.0), axis=1, keepdims=True)
    v1 = 1.0 / z
    v2 = jnp.exp(m2 - m1) / z
    ids_ref[...] = jnp.concatenate([i1, i2], axis=1).T
    vals_ref[...] = jnp.concatenate([v1, v2], axis=1).T


_gate = pl.pallas_call(
    _gate_body,
    out_shape=[
        jax.ShapeDtypeStruct((2, N), jnp.int32),
        jax.ShapeDtypeStruct((2, N), jnp.float32),
    ],
)


def _route_body(ids_hbm, vals_hbm, x_hbm, disp_hbm, dest_hbm, gates_hbm,
                ids_v, vals_v, dest_v, gates_v, src_v, rows_a, rows_b, hist_v,
                gsem, wsem):
    wid = lax.axis_index("s") * NC + lax.axis_index("c")
    base = wid * CHUNK
    pltpu.sync_copy(ids_hbm, ids_v)
    pltpu.sync_copy(vals_hbm.at[pl.ds(base, CHUNK)], vals_v)
    iota = lax.iota(jnp.int32, 16)

    hist_v[pl.ds(0, 16)] = jnp.zeros(16, jnp.int32)
    ones = jnp.full((16,), 1, jnp.int32)

    def _pref(v, _):
        plsc.addupdate_scatter(hist_v, [ids_v[pl.ds(v * 16, 16)]], ones)
        return 0

    lax.fori_loop(0, wid * 16, _pref, 0)
    hv = hist_v[pl.ds(0, 16)]
    h = [jnp.sum(jnp.where(iota == e, hv, 0)) for e in range(E)]
    for v in range(CHUNK // 16):
        idsv = ids_v[pl.ds(base + v * 16, 16)]
        loc = jnp.zeros(16, jnp.int32)
        for e in range(E):
            m = idsv == e
            mi = jnp.where(m, 1, 0)
            cs = plsc.cumsum(mi)
            loc = jnp.where(m, cs - 1 + _bc(h[e]), loc)
            h[e] = h[e] + jnp.sum(mi)
        within = loc < C
        gates_v[pl.ds(v * 16, 16)] = jnp.where(within, vals_v[pl.ds(v * 16, 16)], 0.0)
        dest_v[pl.ds(v * 16, 16)] = idsv * C + jnp.minimum(loc, C - 1)
    pltpu.sync_copy(dest_v, dest_hbm.at[pl.ds(base, CHUNK)])
    pltpu.sync_copy(gates_v, gates_hbm.at[pl.ds(base, CHUNK)])

    em = wid // WPE
    lo = (wid % WPE) * CHUNK
    for v in range(CHUNK // 16):
        src_v[pl.ds(v * 16, 16)] = jnp.zeros(16, jnp.int32)

    def _slots(v, cnt):
        idsv = ids_v[pl.ds(v * 16, 16)]
        m = idsv == _bc(em)
        mi = jnp.where(m, 1, 0)
        pos = _bc(cnt) + plsc.cumsum(mi) - 1
        sel = m & (pos >= _bc(lo)) & (pos < _bc(lo + CHUNK))
        tok = (_bc(v * 16) + iota) & (N - 1)
        idx = jnp.clip(pos - lo, 0, CHUNK - 1)
        plsc.store_scatter(src_v, [idx], tok, mask=sel)
        return cnt + jnp.sum(mi)

    lax.fori_loop(0, A // 16, _slots, jnp.int32(0))

    slot0 = em * C + lo
    bufs = [rows_a, rows_b]
    gd = [None] * NR
    wd = [None] * NR
    for r in range(min(2, NR)):
        gd[r] = pltpu.async_copy(x_hbm.at[src_v.at[pl.ds(r * GR, GR)]],
                                 bufs[r % 2], gsem)
    for r in range(NR):
        gd[r].wait()
        wd[r] = pltpu.async_copy(bufs[r % 2],
                                 disp_hbm.at[pl.ds(slot0 + r * GR, GR)], wsem)
        if r + 2 < NR:
            wd[r].wait()
            gd[r + 2] = pltpu.async_copy(x_hbm.at[src_v.at[pl.ds((r + 2) * GR, GR)]],
                                         bufs[r % 2], gsem)
    for r in range(max(0, NR - 2), NR):
        wd[r].wait()


_route = pl.kernel(
    _route_body,
    out_type=[
        jax.ShapeDtypeStruct((A, D), jnp.float32),
        jax.ShapeDtypeStruct((A,), jnp.int32),
        jax.ShapeDtypeStruct((A,), jnp.float32),
    ],
    mesh=plsc.VectorSubcoreMesh(core_axis_name="c", subcore_axis_name="s",
                                num_cores=NC, num_subcores=NS),
    compiler_params=pltpu.CompilerParams(needs_layout_passes=False),
    scratch_types=[
        pltpu.VMEM((A,), jnp.int32),
        pltpu.VMEM((CHUNK,), jnp.float32),
        pltpu.VMEM((CHUNK,), jnp.int32),
        pltpu.VMEM((CHUNK,), jnp.float32),
        pltpu.VMEM((CHUNK,), jnp.int32),
        pltpu.VMEM((GR, D), jnp.float32),
        pltpu.VMEM((GR, D), jnp.float32),
        pltpu.VMEM((16,), jnp.int32),
        pltpu.SemaphoreType.DMA,
        pltpu.SemaphoreType.DMA,
    ],
)


CB = 1024

def _ffn_body(disp_ref, w1_ref, b1_ref, w2_ref, b2_ref, y_ref):
    a = disp_ref[0].astype(jnp.bfloat16)
    h = jnp.dot(a, w1_ref[0].astype(jnp.bfloat16), preferred_element_type=jnp.float32)
    h = jnp.maximum(h + b1_ref[0], 0.0)
    y = jnp.dot(h.astype(jnp.bfloat16), w2_ref[0].astype(jnp.bfloat16),
                preferred_element_type=jnp.float32)
    y_ref[0] = y + b2_ref[0]


_ffn = pl.pallas_call(
    _ffn_body,
    grid=(E, C // CB),
    in_specs=[
        pl.BlockSpec((1, CB, D), lambda e, c: (e, c, 0)),
        pl.BlockSpec((1, D, D), lambda e, c: (e, 0, 0)),
        pl.BlockSpec((1, 1, D), lambda e, c: (e, 0, 0)),
        pl.BlockSpec((1, D, D), lambda e, c: (e, 0, 0)),
        pl.BlockSpec((1, 1, D), lambda e, c: (e, 0, 0)),
    ],
    out_specs=pl.BlockSpec((1, CB, D), lambda e, c: (e, c, 0)),
    out_shape=jax.ShapeDtypeStruct((E, C, D), jnp.float32),
)


def _combine_body(y_hbm, dest_hbm, gates_hbm, out_hbm,
                  idx_v, g_v, r0a, r1a, r0b, r1b, oba, obb, gsem, wsem):
    wid = lax.axis_index("s") * NC + lax.axis_index("c")
    iota = lax.iota(jnp.int32, 16)
    base0 = wid * TPB
    pltpu.sync_copy(dest_hbm.at[pl.ds(base0, TPB)], idx_v.at[pl.ds(0, TPB)])
    pltpu.sync_copy(dest_hbm.at[pl.ds(N + base0, TPB)], idx_v.at[pl.ds(TPB, TPB)])
    pltpu.sync_copy(gates_hbm.at[pl.ds(base0, TPB)], g_v.at[pl.ds(0, TPB)])
    pltpu.sync_copy(gates_hbm.at[pl.ds(N + base0, TPB)], g_v.at[pl.ds(TPB, TPB)])

    r0 = [r0a, r0b]
    r1 = [r1a, r1b]
    obs = [oba, obb]
    gd0 = [None] * CRND
    gd1 = [None] * CRND
    wd = [None] * CRND

    def _fire(r):
        b = r % 2
        gd0[r] = pltpu.async_copy(y_hbm.at[idx_v.at[pl.ds(r * RT, RT)]], r0[b], gsem)
        gd1[r] = pltpu.async_copy(y_hbm.at[idx_v.at[pl.ds(TPB + r * RT, RT)]], r1[b], gsem)

    _fire(0)
    if CRND > 1:
        _fire(1)
    for r in range(CRND):
        b = r % 2
        gd0[r].wait()
        gd1[r].wait()
        if r >= 2:
            wd[r - 2].wait()
        rows0, rows1, ob = r0[b], r1[b], obs[b]
        g0c = g_v[pl.ds(r * RT, 16)]
        g1c = g_v[pl.ds(TPB + r * RT, 16)]

        def _tok(t, _):
            ln = _bc(t)
            g0v = _bc(jnp.sum(jnp.where(iota == ln, g0c, 0.0)))
            g1v = _bc(jnp.sum(jnp.where(iota == ln, g1c, 0.0)))

            def _j(j, __):
                ob[t, pl.ds(j * 16, 16)] = (g0v * rows0[t, pl.ds(j * 16, 16)]
                                            + g1v * rows1[t, pl.ds(j * 16, 16)])
                return 0

            lax.fori_loop(0, D // 16, _j, 0)
            return 0

        lax.fori_loop(0, RT, _tok, 0)
        wd[r] = pltpu.async_copy(ob, out_hbm.at[pl.ds(base0 + r * RT, RT)], wsem)
        if r + 2 < CRND:
            _fire(r + 2)
    for r in range(max(0, CRND - 2), CRND):
        wd[r].wait()


_combine = pl.kernel(
    _combine_body,
    out_type=jax.ShapeDtypeStruct((N, D), jnp.float32),
    mesh=plsc.VectorSubcoreMesh(core_axis_name="c", subcore_axis_name="s",
                                num_cores=NC, num_subcores=NS),
    compiler_params=pltpu.CompilerParams(needs_layout_passes=False),
    scratch_types=[
        pltpu.VMEM((2 * TPB,), jnp.int32),
        pltpu.VMEM((2 * TPB,), jnp.float32),
        pltpu.VMEM((RT, D), jnp.float32),
        pltpu.VMEM((RT, D), jnp.float32),
        pltpu.VMEM((RT, D), jnp.float32),
        pltpu.VMEM((RT, D), jnp.float32),
        pltpu.VMEM((RT, D), jnp.float32),
        pltpu.VMEM((RT, D), jnp.float32),
        pltpu.SemaphoreType.DMA,
        pltpu.SemaphoreType.DMA,
    ],
)


def kernel(input, wg, w1, b1, w2, b2):
    x = input.astype(jnp.float32)
    wgp = jnp.zeros((D, 128), jnp.float32).at[:, :E].set(wg.T)
    ids2, vals2 = _gate(x, wgp)
    disp, dest, gates = _route(ids2.reshape(A), vals2.reshape(A), x)
    y = _ffn(disp.reshape(E, C, D), w1, b1, w2, b2)
    out = _combine(y.reshape(A, D), dest, gates)
    return out


# speedup vs baseline: 2.5236x; 1.0116x over previous
import jax
import jax.numpy as jnp
from jax import lax
from jax.experimental import pallas as pl
from jax.experimental.pallas import tpu as pltpu
from jax.experimental.pallas import tpu_sc as plsc

E = 8
K = 2
D = 1024
N = 4096
C = 1024
A = K * N
NC, NS = 2, 16
NW = NC * NS
CHUNK = A // NW
WPE = C // CHUNK
GR = 32
NR = CHUNK // GR
TPB = N // NW
RT = 16
CRND = TPB // RT
NEG = -1e30


def _bc(s):
    return lax.broadcast(s, (16,))


def _gate_body(x_ref, wg_ref, ids_ref, vals_ref):
    lg = jnp.dot(x_ref[...], wg_ref[...], preferred_element_type=jnp.float32)
    col = lax.broadcasted_iota(jnp.int32, lg.shape, 1)
    lg = jnp.where(col < E, lg, NEG)
    m1 = jnp.max(lg, axis=1, keepdims=True)
    i1 = jnp.min(jnp.where(lg >= m1, col, 128), axis=1, keepdims=True)
    lg2 = jnp.where(col == i1, NEG, lg)
    m2 = jnp.max(lg2, axis=1, keepdims=True)
    i2 = jnp.min(jnp.where(lg2 >= m2, col, 128), axis=1, keepdims=True)
    z = jnp.sum(jnp.where(col < E, jnp.exp(lg - m1), 0.0), axis=1, keepdims=True)
    v1 = 1.0 / z
    v2 = jnp.exp(m2 - m1) / z
    ids_ref[...] = jnp.concatenate([i1, i2], axis=1).T
    vals_ref[...] = jnp.concatenate([v1, v2], axis=1).T


_gate = pl.pallas_call(
    _gate_body,
    out_shape=[
        jax.ShapeDtypeStruct((2, N), jnp.int32),
        jax.ShapeDtypeStruct((2, N), jnp.float32),
    ],
)


def _route_body(ids_hbm, vals_hbm, x_hbm, disp_hbm, dest_hbm, gates_hbm,
                ids_v, vals_v, dest_v, gates_v, src_v, rows_a, rows_b, hist_v,
                gsem, wsem):
    wid = lax.axis_index("s") * NC + lax.axis_index("c")
    base = wid * CHUNK
    pltpu.sync_copy(ids_hbm.at[0], ids_v.at[pl.ds(0, N)])
    pltpu.sync_copy(ids_hbm.at[1], ids_v.at[pl.ds(N, N)])
    kk = wid // (N // CHUNK)
    off = base - kk * N
    pltpu.sync_copy(vals_hbm.at[kk, pl.ds(off, CHUNK)], vals_v)
    iota = lax.iota(jnp.int32, 16)

    hist_v[pl.ds(0, 16)] = jnp.zeros(16, jnp.int32)
    ones = jnp.full((16,), 1, jnp.int32)

    def _pref(v, _):
        plsc.addupdate_scatter(hist_v, [ids_v[pl.ds(v * 16, 16)]], ones)
        return 0

    lax.fori_loop(0, wid * 16, _pref, 0)
    hv = hist_v[pl.ds(0, 16)]
    h = [jnp.sum(jnp.where(iota == e, hv, 0)) for e in range(E)]
    for v in range(CHUNK // 16):
        idsv = ids_v[pl.ds(base + v * 16, 16)]
        loc = jnp.zeros(16, jnp.int32)
        for e in range(E):
            m = idsv == e
            mi = jnp.where(m, 1, 0)
            cs = plsc.cumsum(mi)
            loc = jnp.where(m, cs - 1 + _bc(h[e]), loc)
            h[e] = h[e] + jnp.sum(mi)
        within = loc < C
        gates_v[pl.ds(v * 16, 16)] = jnp.where(within, vals_v[pl.ds(v * 16, 16)], 0.0)
        dest_v[pl.ds(v * 16, 16)] = idsv * C + jnp.minimum(loc, C - 1)
    pltpu.sync_copy(dest_v, dest_hbm.at[pl.ds(base, CHUNK)])
    pltpu.sync_copy(gates_v, gates_hbm.at[pl.ds(base, CHUNK)])

    em = wid // WPE
    lo = (wid % WPE) * CHUNK
    for v in range(CHUNK // 16):
        src_v[pl.ds(v * 16, 16)] = jnp.zeros(16, jnp.int32)

    def _slots(v, cnt):
        idsv = ids_v[pl.ds(v * 16, 16)]
        m = idsv == _bc(em)
        mi = jnp.where(m, 1, 0)
        pos = _bc(cnt) + plsc.cumsum(mi) - 1
        sel = m & (pos >= _bc(lo)) & (pos < _bc(lo + CHUNK))
        tok = (_bc(v * 16) + iota) & (N - 1)
        idx = jnp.clip(pos - lo, 0, CHUNK - 1)
        plsc.store_scatter(src_v, [idx], tok, mask=sel)
        return cnt + jnp.sum(mi)

    lax.fori_loop(0, A // 16, _slots, jnp.int32(0))

    slot0 = em * C + lo
    bufs = [rows_a, rows_b]
    gd = [None] * NR
    wd = [None] * NR
    for r in range(min(2, NR)):
        gd[r] = pltpu.async_copy(x_hbm.at[src_v.at[pl.ds(r * GR, GR)]],
                                 bufs[r % 2], gsem)
    for r in range(NR):
        gd[r].wait()
        wd[r] = pltpu.async_copy(bufs[r % 2],
                                 disp_hbm.at[pl.ds(slot0 + r * GR, GR)], wsem)
        if r + 2 < NR:
            wd[r].wait()
            gd[r + 2] = pltpu.async_copy(x_hbm.at[src_v.at[pl.ds((r + 2) * GR, GR)]],
                                         bufs[r % 2], gsem)
    for r in range(max(0, NR - 2), NR):
        wd[r].wait()


_route = pl.kernel(
    _route_body,
    out_type=[
        jax.ShapeDtypeStruct((A, D), jnp.float32),
        jax.ShapeDtypeStruct((A,), jnp.int32),
        jax.ShapeDtypeStruct((A,), jnp.float32),
    ],
    mesh=plsc.VectorSubcoreMesh(core_axis_name="c", subcore_axis_name="s",
                                num_cores=NC, num_subcores=NS),
    compiler_params=pltpu.CompilerParams(needs_layout_passes=False),
    name="route_sc",
    scratch_types=[
        pltpu.VMEM((A,), jnp.int32),
        pltpu.VMEM((CHUNK,), jnp.float32),
        pltpu.VMEM((CHUNK,), jnp.int32),
        pltpu.VMEM((CHUNK,), jnp.float32),
        pltpu.VMEM((CHUNK,), jnp.int32),
        pltpu.VMEM((GR, D), jnp.float32),
        pltpu.VMEM((GR, D), jnp.float32),
        pltpu.VMEM((16,), jnp.int32),
        pltpu.SemaphoreType.DMA,
        pltpu.SemaphoreType.DMA,
    ],
)


CB = 1024

def _ffn_body(disp_ref, w1_ref, b1_ref, w2_ref, b2_ref, y_ref):
    a = disp_ref[0].astype(jnp.bfloat16)
    h = jnp.dot(a, w1_ref[0].astype(jnp.bfloat16), preferred_element_type=jnp.float32)
    h = jnp.maximum(h + b1_ref[0], 0.0)
    y = jnp.dot(h.astype(jnp.bfloat16), w2_ref[0].astype(jnp.bfloat16),
                preferred_element_type=jnp.float32)
    y_ref[0] = y + b2_ref[0]


_ffn = pl.pallas_call(
    _ffn_body,
    grid=(E, C // CB),
    in_specs=[
        pl.BlockSpec((1, CB, D), lambda e, c: (e, c, 0)),
        pl.BlockSpec((1, D, D), lambda e, c: (e, 0, 0)),
        pl.BlockSpec((1, 1, D), lambda e, c: (e, 0, 0)),
        pl.BlockSpec((1, D, D), lambda e, c: (e, 0, 0)),
        pl.BlockSpec((1, 1, D), lambda e, c: (e, 0, 0)),
    ],
    out_specs=pl.BlockSpec((1, CB, D), lambda e, c: (e, c, 0)),
    out_shape=jax.ShapeDtypeStruct((E, C, D), jnp.float32),
)


def _combine_body(y_hbm, dest_hbm, gates_hbm, out_hbm,
                  idx_v, g_v, r0a, r1a, r0b, r1b, oba, obb, gsem, wsem):
    wid = lax.axis_index("s") * NC + lax.axis_index("c")
    iota = lax.iota(jnp.int32, 16)
    base0 = wid * TPB
    pltpu.sync_copy(dest_hbm.at[pl.ds(base0, TPB)], idx_v.at[pl.ds(0, TPB)])
    pltpu.sync_copy(dest_hbm.at[pl.ds(N + base0, TPB)], idx_v.at[pl.ds(TPB, TPB)])
    pltpu.sync_copy(gates_hbm.at[pl.ds(base0, TPB)], g_v.at[pl.ds(0, TPB)])
    pltpu.sync_copy(gates_hbm.at[pl.ds(N + base0, TPB)], g_v.at[pl.ds(TPB, TPB)])

    r0 = [r0a, r0b]
    r1 = [r1a, r1b]
    obs = [oba, obb]
    gd0 = [None] * CRND
    gd1 = [None] * CRND
    wd = [None] * CRND

    def _fire(r):
        b = r % 2
        gd0[r] = pltpu.async_copy(y_hbm.at[idx_v.at[pl.ds(r * RT, RT)]], r0[b], gsem)
        gd1[r] = pltpu.async_copy(y_hbm.at[idx_v.at[pl.ds(TPB + r * RT, RT)]], r1[b], gsem)

    _fire(0)
    if CRND > 1:
        _fire(1)
    for r in range(CRND):
        b = r % 2
        gd0[r].wait()
        gd1[r].wait()
        if r >= 2:
            wd[r - 2].wait()
        rows0, rows1, ob = r0[b], r1[b], obs[b]
        g0c = g_v[pl.ds(r * RT, 16)]
        g1c = g_v[pl.ds(TPB + r * RT, 16)]

        def _tok(t, _):
            ln = _bc(t)
            g0v = _bc(jnp.sum(jnp.where(iota == ln, g0c, 0.0)))
            g1v = _bc(jnp.sum(jnp.where(iota == ln, g1c, 0.0)))

            def _j(j, __):
                ob[t, pl.ds(j * 16, 16)] = (g0v * rows0[t, pl.ds(j * 16, 16)]
                                            + g1v * rows1[t, pl.ds(j * 16, 16)])
                return 0

            lax.fori_loop(0, D // 16, _j, 0)
            return 0

        lax.fori_loop(0, RT, _tok, 0)
        wd[r] = pltpu.async_copy(ob, out_hbm.at[pl.ds(base0 + r * RT, RT)], wsem)
        if r + 2 < CRND:
            _fire(r + 2)
    for r in range(max(0, CRND - 2), CRND):
        wd[r].wait()


_combine = pl.kernel(
    _combine_body,
    out_type=jax.ShapeDtypeStruct((N, D), jnp.float32),
    mesh=plsc.VectorSubcoreMesh(core_axis_name="c", subcore_axis_name="s",
                                num_cores=NC, num_subcores=NS),
    compiler_params=pltpu.CompilerParams(needs_layout_passes=False),
    scratch_types=[
        pltpu.VMEM((2 * TPB,), jnp.int32),
        pltpu.VMEM((2 * TPB,), jnp.float32),
        pltpu.VMEM((RT, D), jnp.float32),
        pltpu.VMEM((RT, D), jnp.float32),
        pltpu.VMEM((RT, D), jnp.float32),
        pltpu.VMEM((RT, D), jnp.float32),
        pltpu.VMEM((RT, D), jnp.float32),
        pltpu.VMEM((RT, D), jnp.float32),
        pltpu.SemaphoreType.DMA,
        pltpu.SemaphoreType.DMA,
    ],
)


def kernel(input, wg, w1, b1, w2, b2):
    x = input.astype(jnp.float32)
    wgp = jnp.zeros((D, 128), jnp.float32).at[:, :E].set(wg.T)
    ids2, vals2 = _gate(x, wgp)
    disp, dest, gates = _route(ids2, vals2, x)
    y = _ffn(disp.reshape(E, C, D), w1, b1, w2, b2)
    out = _combine(y.reshape(A, D), dest, gates)
    return out


# speedup vs baseline: 2.5551x; 1.0125x over previous
import jax
import jax.numpy as jnp
from jax import lax
from jax.experimental import pallas as pl
from jax.experimental.pallas import tpu as pltpu
from jax.experimental.pallas import tpu_sc as plsc

E = 8
K = 2
D = 1024
N = 4096
C = 1024
A = K * N
NC, NS = 2, 16
NW = NC * NS
CHUNK = A // NW
WPE = C // CHUNK
GR = 32
NR = CHUNK // GR
TPB = N // NW
RT = 8
CRND = TPB // RT
NEG = -1e30


def _bc(s):
    return lax.broadcast(s, (16,))


def _gate_body(x_ref, wg_ref, ids_ref, vals_ref):
    lg = jnp.dot(x_ref[...], wg_ref[...], preferred_element_type=jnp.float32)
    col = lax.broadcasted_iota(jnp.int32, lg.shape, 1)
    lg = jnp.where(col < E, lg, NEG)
    m1 = jnp.max(lg, axis=1, keepdims=True)
    i1 = jnp.min(jnp.where(lg >= m1, col, 128), axis=1, keepdims=True)
    lg2 = jnp.where(col == i1, NEG, lg)
    m2 = jnp.max(lg2, axis=1, keepdims=True)
    i2 = jnp.min(jnp.where(lg2 >= m2, col, 128), axis=1, keepdims=True)
    z = jnp.sum(jnp.where(col < E, jnp.exp(lg - m1), 0.0), axis=1, keepdims=True)
    v1 = 1.0 / z
    v2 = jnp.exp(m2 - m1) / z
    ids_ref[...] = jnp.concatenate([i1, i2], axis=1).T
    vals_ref[...] = jnp.concatenate([v1, v2], axis=1).T


_gate = pl.pallas_call(
    _gate_body,
    out_shape=[
        jax.ShapeDtypeStruct((2, N), jnp.int32),
        jax.ShapeDtypeStruct((2, N), jnp.float32),
    ],
)


def _route_body(ids_hbm, vals_hbm, x_hbm, disp_hbm, dest_hbm, gates_hbm,
                ids_v, vals_v, dest_v, gates_v, src_v, rows_a, rows_b, hist_v,
                gsem, wsem):
    wid = lax.axis_index("s") * NC + lax.axis_index("c")
    base = wid * CHUNK
    pltpu.sync_copy(ids_hbm.at[0], ids_v.at[pl.ds(0, N)])
    pltpu.sync_copy(ids_hbm.at[1], ids_v.at[pl.ds(N, N)])
    kk = wid // (N // CHUNK)
    off = base - kk * N
    pltpu.sync_copy(vals_hbm.at[kk, pl.ds(off, CHUNK)], vals_v)
    iota = lax.iota(jnp.int32, 16)

    em = wid // WPE
    lo = (wid % WPE) * CHUNK
    for v in range(CHUNK // 16):
        src_v[pl.ds(v * 16, 16)] = jnp.zeros(16, jnp.int32)

    def _slots(v, cnt):
        idsv = ids_v[pl.ds(v * 16, 16)]
        m = idsv == _bc(em)
        mi = jnp.where(m, 1, 0)
        pos = _bc(cnt) + plsc.cumsum(mi) - 1
        sel = m & (pos >= _bc(lo)) & (pos < _bc(lo + CHUNK))
        tok = (_bc(v * 16) + iota) & (N - 1)
        idx = jnp.clip(pos - lo, 0, CHUNK - 1)
        plsc.store_scatter(src_v, [idx], tok, mask=sel)
        return cnt + jnp.sum(mi)

    lax.fori_loop(0, A // 16, _slots, jnp.int32(0))

    slot0 = em * C + lo
    bufs = [rows_a, rows_b]
    gd = [None] * NR
    wd = [None] * NR
    for r in range(min(2, NR)):
        gd[r] = pltpu.async_copy(x_hbm.at[src_v.at[pl.ds(r * GR, GR)]],
                                 bufs[r % 2], gsem)

    hist_v[pl.ds(0, 16)] = jnp.zeros(16, jnp.int32)
    ones = jnp.full((16,), 1, jnp.int32)

    def _pref(v, _):
        plsc.addupdate_scatter(hist_v, [ids_v[pl.ds(v * 16, 16)]], ones)
        return 0

    lax.fori_loop(0, wid * 16, _pref, 0)
    hv = hist_v[pl.ds(0, 16)]
    h = [jnp.sum(jnp.where(iota == e, hv, 0)) for e in range(E)]
    for v in range(CHUNK // 16):
        idsv = ids_v[pl.ds(base + v * 16, 16)]
        loc = jnp.zeros(16, jnp.int32)
        for e in range(E):
            m = idsv == e
            mi = jnp.where(m, 1, 0)
            cs = plsc.cumsum(mi)
            loc = jnp.where(m, cs - 1 + _bc(h[e]), loc)
            h[e] = h[e] + jnp.sum(mi)
        within = loc < C
        gates_v[pl.ds(v * 16, 16)] = jnp.where(within, vals_v[pl.ds(v * 16, 16)], 0.0)
        dest_v[pl.ds(v * 16, 16)] = idsv * C + jnp.minimum(loc, C - 1)
    pltpu.sync_copy(dest_v, dest_hbm.at[pl.ds(base, CHUNK)])
    pltpu.sync_copy(gates_v, gates_hbm.at[pl.ds(base, CHUNK)])

    for r in range(NR):
        gd[r].wait()
        wd[r] = pltpu.async_copy(bufs[r % 2],
                                 disp_hbm.at[pl.ds(slot0 + r * GR, GR)], wsem)
        if r + 2 < NR:
            wd[r].wait()
            gd[r + 2] = pltpu.async_copy(x_hbm.at[src_v.at[pl.ds((r + 2) * GR, GR)]],
                                         bufs[r % 2], gsem)
    for r in range(max(0, NR - 2), NR):
        wd[r].wait()


_route = pl.kernel(
    _route_body,
    out_type=[
        jax.ShapeDtypeStruct((A, D), jnp.float32),
        jax.ShapeDtypeStruct((A,), jnp.int32),
        jax.ShapeDtypeStruct((A,), jnp.float32),
    ],
    mesh=plsc.VectorSubcoreMesh(core_axis_name="c", subcore_axis_name="s",
                                num_cores=NC, num_subcores=NS),
    compiler_params=pltpu.CompilerParams(needs_layout_passes=False),
    name="route_sc",
    scratch_types=[
        pltpu.VMEM((A,), jnp.int32),
        pltpu.VMEM((CHUNK,), jnp.float32),
        pltpu.VMEM((CHUNK,), jnp.int32),
        pltpu.VMEM((CHUNK,), jnp.float32),
        pltpu.VMEM((CHUNK,), jnp.int32),
        pltpu.VMEM((GR, D), jnp.float32),
        pltpu.VMEM((GR, D), jnp.float32),
        pltpu.VMEM((16,), jnp.int32),
        pltpu.SemaphoreType.DMA,
        pltpu.SemaphoreType.DMA,
    ],
)


CB = 1024

def _ffn_body(disp_ref, w1_ref, b1_ref, w2_ref, b2_ref, y_ref):
    a = disp_ref[0].astype(jnp.bfloat16)
    h = jnp.dot(a, w1_ref[0].astype(jnp.bfloat16), preferred_element_type=jnp.float32)
    h = jnp.maximum(h + b1_ref[0], 0.0)
    y = jnp.dot(h.astype(jnp.bfloat16), w2_ref[0].astype(jnp.bfloat16),
                preferred_element_type=jnp.float32)
    y_ref[0] = y + b2_ref[0]


_ffn = pl.pallas_call(
    _ffn_body,
    grid=(E, C // CB),
    in_specs=[
        pl.BlockSpec((1, CB, D), lambda e, c: (e, c, 0)),
        pl.BlockSpec((1, D, D), lambda e, c: (e, 0, 0)),
        pl.BlockSpec((1, 1, D), lambda e, c: (e, 0, 0)),
        pl.BlockSpec((1, D, D), lambda e, c: (e, 0, 0)),
        pl.BlockSpec((1, 1, D), lambda e, c: (e, 0, 0)),
    ],
    out_specs=pl.BlockSpec((1, CB, D), lambda e, c: (e, c, 0)),
    out_shape=jax.ShapeDtypeStruct((E, C, D), jnp.float32),
)


def _combine_body(y_hbm, dest_hbm, gates_hbm, out_hbm,
                  idx_v, g_v, r0a, r1a, r0b, r1b, r0c, r1c, oba, obb,
                  gsem, wsem):
    wid = lax.axis_index("s") * NC + lax.axis_index("c")
    iota = lax.iota(jnp.int32, 16)
    base0 = wid * TPB
    pltpu.sync_copy(dest_hbm.at[pl.ds(base0, TPB)], idx_v.at[pl.ds(0, TPB)])
    pltpu.sync_copy(dest_hbm.at[pl.ds(N + base0, TPB)], idx_v.at[pl.ds(TPB, TPB)])
    pltpu.sync_copy(gates_hbm.at[pl.ds(base0, TPB)], g_v.at[pl.ds(0, TPB)])
    pltpu.sync_copy(gates_hbm.at[pl.ds(N + base0, TPB)], g_v.at[pl.ds(TPB, TPB)])

    r0 = [r0a, r0b, r0c]
    r1 = [r1a, r1b, r1c]
    obs = [oba, obb]
    gd0 = [None] * CRND
    gd1 = [None] * CRND
    wd = [None] * CRND

    def _fire(r):
        b = r % 3
        gd0[r] = pltpu.async_copy(y_hbm.at[idx_v.at[pl.ds(r * RT, RT)]], r0[b], gsem)
        gd1[r] = pltpu.async_copy(y_hbm.at[idx_v.at[pl.ds(TPB + r * RT, RT)]], r1[b], gsem)

    for r in range(min(3, CRND)):
        _fire(r)
    for r in range(CRND):
        b = r % 3
        gd0[r].wait()
        gd1[r].wait()
        if r >= 2:
            wd[r - 2].wait()
        rows0, rows1, ob = r0[b], r1[b], obs[r % 2]
        g0c = g_v[pl.ds(r * RT, 16)]
        g1c = g_v[pl.ds(TPB + r * RT, 16)]

        def _tok(t, _):
            ln = _bc(t)
            g0v = _bc(jnp.sum(jnp.where(iota == ln, g0c, 0.0)))
            g1v = _bc(jnp.sum(jnp.where(iota == ln, g1c, 0.0)))

            def _j(j, __):
                ob[t, pl.ds(j * 16, 16)] = (g0v * rows0[t, pl.ds(j * 16, 16)]
                                            + g1v * rows1[t, pl.ds(j * 16, 16)])
                return 0

            lax.fori_loop(0, D // 16, _j, 0)
            return 0

        lax.fori_loop(0, RT, _tok, 0)
        wd[r] = pltpu.async_copy(ob, out_hbm.at[pl.ds(base0 + r * RT, RT)], wsem)
        if r + 3 < CRND:
            _fire(r + 3)
    for r in range(max(0, CRND - 2), CRND):
        wd[r].wait()


_combine = pl.kernel(
    _combine_body,
    out_type=jax.ShapeDtypeStruct((N, D), jnp.float32),
    mesh=plsc.VectorSubcoreMesh(core_axis_name="c", subcore_axis_name="s",
                                num_cores=NC, num_subcores=NS),
    compiler_params=pltpu.CompilerParams(needs_layout_passes=False),
    name="combine_sc",
    scratch_types=[
        pltpu.VMEM((2 * TPB + 16,), jnp.int32),
        pltpu.VMEM((2 * TPB + 16,), jnp.float32),
        pltpu.VMEM((RT, D), jnp.float32),
        pltpu.VMEM((RT, D), jnp.float32),
        pltpu.VMEM((RT, D), jnp.float32),
        pltpu.VMEM((RT, D), jnp.float32),
        pltpu.VMEM((RT, D), jnp.float32),
        pltpu.VMEM((RT, D), jnp.float32),
        pltpu.VMEM((RT, D), jnp.float32),
        pltpu.VMEM((RT, D), jnp.float32),
        pltpu.SemaphoreType.DMA,
        pltpu.SemaphoreType.DMA,
    ],
)


def kernel(input, wg, w1, b1, w2, b2):
    x = input.astype(jnp.float32)
    wgp = jnp.zeros((D, 128), jnp.float32).at[:, :E].set(wg.T)
    ids2, vals2 = _gate(x, wgp)
    disp, dest, gates = _route(ids2, vals2, x)
    y = _ffn(disp.reshape(E, C, D), w1, b1, w2, b2)
    out = _combine(y.reshape(A, D), dest, gates)
    return out
